# Initial kernel scaffold; baseline (speedup 1.0000x reference)
import jax
import jax.numpy as jnp
from jax.experimental import pallas as pl


def kernel(mask, weight):
    raise NotImplementedError("write your pallas kernel here")


# speedup vs baseline: 1.7956x; 1.7956x over previous
import functools

import jax
import jax.numpy as jnp
from jax import lax
from jax.experimental import pallas as pl
from jax.experimental.pallas import tpu as pltpu
from jax.experimental.pallas import tpu_sc as plsc

_NUM_CORES = 2
_NUM_SUBCORES = 16
_NUM_WORKERS = _NUM_CORES * _NUM_SUBCORES
_CHUNK = 512


def _sc_gather(weight, idx, num_indices, dim):
    b_per_w = num_indices // _NUM_WORKERS
    steps = b_per_w // _CHUNK
    mesh = plsc.VectorSubcoreMesh(core_axis_name="c", subcore_axis_name="s")

    @functools.partial(
        pl.kernel,
        mesh=mesh,
        compiler_params=pltpu.CompilerParams(use_tc_tiling_on_sc=False),
        out_type=jax.ShapeDtypeStruct((num_indices, dim), jnp.float32),
        scratch_types=[
            pltpu.VMEM((_CHUNK,), jnp.int32),
            pltpu.VMEM((_CHUNK, dim), jnp.float32),
            pltpu.SemaphoreType.DMA,
        ],
    )
    def k(table_hbm, idx_hbm, out_hbm, idx_v, rows_v, sem):
        wid = lax.axis_index("s") * _NUM_CORES + lax.axis_index("c")
        base = wid * b_per_w

        @pl.loop(0, steps)
        def _(i):
            off = base + i * _CHUNK
            pltpu.sync_copy(idx_hbm.at[pl.ds(off, _CHUNK)], idx_v)
            pltpu.async_copy(table_hbm.at[idx_v], rows_v, sem).wait()
            pltpu.sync_copy(rows_v, out_hbm.at[pl.ds(off, _CHUNK)])

    return k(weight, idx)


def kernel(mask, weight):
    batch, hist = mask.shape
    _, dim = weight.shape
    num_indices = batch * hist
    idx = mask.reshape(num_indices)
    out = _sc_gather(weight, idx, num_indices, dim)
    return out.reshape(batch, hist, dim)


# speedup vs baseline: 1.8701x; 1.0415x over previous
import functools

import jax
import jax.numpy as jnp
from jax import lax
from jax.experimental import pallas as pl
from jax.experimental.pallas import tpu as pltpu
from jax.experimental.pallas import tpu_sc as plsc

_NUM_CORES = 2
_NUM_SUBCORES = 16
_NUM_WORKERS = _NUM_CORES * _NUM_SUBCORES
_CHUNK = 512


def _sc_gather(weight, idx, num_indices, dim):
    b_per_w = num_indices // _NUM_WORKERS
    steps = b_per_w // _CHUNK
    mesh = plsc.VectorSubcoreMesh(core_axis_name="c", subcore_axis_name="s")

    @functools.partial(
        pl.kernel,
        mesh=mesh,
        compiler_params=pltpu.CompilerParams(use_tc_tiling_on_sc=False),
        out_type=jax.ShapeDtypeStruct((num_indices, dim), jnp.float32),
        scratch_types=[
            pltpu.VMEM((2, _CHUNK), jnp.int32),
            pltpu.VMEM((2, _CHUNK, dim), jnp.float32),
            pltpu.SemaphoreType.DMA((2,)),
            pltpu.SemaphoreType.DMA((2,)),
            pltpu.SemaphoreType.DMA((2,)),
        ],
    )
    def k(table_hbm, idx_hbm, out_hbm, idx_v, rows_v, sem_i, sem_g, sem_o):
        wid = lax.axis_index("s") * _NUM_CORES + lax.axis_index("c")
        base = wid * b_per_w

        def idx_copy(step, b):
            return pltpu.make_async_copy(
                idx_hbm.at[pl.ds(base + step * _CHUNK, _CHUNK)],
                idx_v.at[b],
                sem_i.at[b],
            )

        def gather_copy(b):
            return pltpu.make_async_copy(
                table_hbm.at[idx_v.at[b]], rows_v.at[b], sem_g.at[b]
            )

        def out_copy(step, b):
            return pltpu.make_async_copy(
                rows_v.at[b],
                out_hbm.at[pl.ds(base + step * _CHUNK, _CHUNK)],
                sem_o.at[b],
            )

        idx_copy(0, 0).start()
        idx_copy(1, 1).start()

        @pl.loop(0, steps, step=2)
        def _(i):
            for b in range(2):
                step = i + b
                idx_copy(step, b).wait()

                @pl.when(step >= 2)
                def _():
                    out_copy(step - 2, b).wait()

                gather_copy(b).start()
                gather_copy(b).wait()
                out_copy(step, b).start()

                @pl.when(step + 2 < steps)
                def _():
                    idx_copy(step + 2, b).start()

        out_copy(steps - 2, 0).wait()
        out_copy(steps - 1, 1).wait()

    return k(weight, idx)


def kernel(mask, weight):
    batch, hist = mask.shape
    _, dim = weight.shape
    num_indices = batch * hist
    idx = mask.reshape(num_indices)
    out = _sc_gather(weight, idx, num_indices, dim)
    return out.reshape(batch, hist, dim)


# speedup vs baseline: 1.8738x; 1.0020x over previous
import functools

import jax
import jax.numpy as jnp
from jax import lax
from jax.experimental import pallas as pl
from jax.experimental.pallas import tpu as pltpu
from jax.experimental.pallas import tpu_sc as plsc

_NUM_CORES = 2
_NUM_SUBCORES = 16
_NUM_WORKERS = _NUM_CORES * _NUM_SUBCORES
_CHUNK = 512


def _sc_gather(weight, idx, num_indices, dim):
    b_per_w = num_indices // _NUM_WORKERS
    steps = b_per_w // _CHUNK
    mesh = plsc.VectorSubcoreMesh(core_axis_name="c", subcore_axis_name="s")

    @functools.partial(
        pl.kernel,
        mesh=mesh,
        compiler_params=pltpu.CompilerParams(use_tc_tiling_on_sc=False),
        out_type=jax.ShapeDtypeStruct((num_indices, dim), jnp.float32),
        scratch_types=[
            pltpu.VMEM((2, _CHUNK), jnp.int32),
            pltpu.VMEM((2, _CHUNK, dim), jnp.float32),
            pltpu.SemaphoreType.DMA((2,)),
            pltpu.SemaphoreType.DMA((2,)),
            pltpu.SemaphoreType.DMA((2,)),
        ],
    )
    def k(table_hbm, idx_hbm, out_hbm, idx_v, rows_v, sem_i, sem_g, sem_o):
        wid = lax.axis_index("s") * _NUM_CORES + lax.axis_index("c")
        base = wid * b_per_w

        def idx_copy(step, b):
            return pltpu.make_async_copy(
                idx_hbm.at[pl.ds(base + step * _CHUNK, _CHUNK)],
                idx_v.at[b],
                sem_i.at[b],
            )

        def gather_copy(b):
            return pltpu.make_async_copy(
                table_hbm.at[idx_v.at[b]], rows_v.at[b], sem_g.at[b]
            )

        def out_copy(step, b):
            return pltpu.make_async_copy(
                rows_v.at[b],
                out_hbm.at[pl.ds(base + step * _CHUNK, _CHUNK)],
                sem_o.at[b],
            )

        idx_copy(0, 0).start()
        idx_copy(1, 1).start()
        idx_copy(0, 0).wait()
        gather_copy(0).start()

        @pl.loop(0, steps, step=2)
        def _(i):
            for b in range(2):
                step = i + b
                b1 = 1 - b

                @pl.when(step + 1 < steps)
                def _():
                    idx_copy(step + 1, b1).wait()

                    @pl.when(step >= 1)
                    def _():
                        out_copy(step - 1, b1).wait()

                    gather_copy(b1).start()

                gather_copy(b).wait()
                out_copy(step, b).start()

                @pl.when(step + 2 < steps)
                def _():
                    idx_copy(step + 2, b).start()

        out_copy(steps - 1, (steps - 1) % 2).wait()

    return k(weight, idx)


def kernel(mask, weight):
    batch, hist = mask.shape
    _, dim = weight.shape
    num_indices = batch * hist
    idx = mask.reshape(num_indices)
    out = _sc_gather(weight, idx, num_indices, dim)
    return out.reshape(batch, hist, dim)


# speedup vs baseline: 1.8739x; 1.0000x over previous
import functools

import jax
import jax.numpy as jnp
from jax import lax
from jax.experimental import pallas as pl
from jax.experimental.pallas import tpu as pltpu
from jax.experimental.pallas import tpu_sc as plsc

_NUM_CORES = 2
_NUM_SUBCORES = 16
_NUM_WORKERS = _NUM_CORES * _NUM_SUBCORES
_CHUNK = 800


def _sc_gather(weight, idx, num_indices, dim):
    b_per_w = num_indices // _NUM_WORKERS
    steps = b_per_w // _CHUNK
    mesh = plsc.VectorSubcoreMesh(core_axis_name="c", subcore_axis_name="s")

    @functools.partial(
        pl.kernel,
        mesh=mesh,
        compiler_params=pltpu.CompilerParams(use_tc_tiling_on_sc=False),
        out_type=jax.ShapeDtypeStruct((num_indices, dim), jnp.float32),
        scratch_types=[
            pltpu.VMEM((2, _CHUNK), jnp.int32),
            pltpu.VMEM((2, _CHUNK, dim), jnp.float32),
            pltpu.SemaphoreType.DMA((2,)),
            pltpu.SemaphoreType.DMA((2,)),
            pltpu.SemaphoreType.DMA((2,)),
        ],
    )
    def k(table_hbm, idx_hbm, out_hbm, idx_v, rows_v, sem_i, sem_g, sem_o):
        wid = lax.axis_index("s") * _NUM_CORES + lax.axis_index("c")
        base = wid * b_per_w

        def idx_copy(step, b):
            return pltpu.make_async_copy(
                idx_hbm.at[pl.ds(base + step * _CHUNK, _CHUNK)],
                idx_v.at[b],
                sem_i.at[b],
            )

        def gather_copy(b):
            return pltpu.make_async_copy(
                table_hbm.at[idx_v.at[b]], rows_v.at[b], sem_g.at[b]
            )

        def out_copy(step, b):
            return pltpu.make_async_copy(
                rows_v.at[b],
                out_hbm.at[pl.ds(base + step * _CHUNK, _CHUNK)],
                sem_o.at[b],
            )

        idx_copy(0, 0).start()
        idx_copy(1, 1).start()
        idx_copy(0, 0).wait()
        gather_copy(0).start()

        @pl.loop(0, steps, step=2)
        def _(i):
            for b in range(2):
                step = i + b
                b1 = 1 - b

                @pl.when(step + 1 < steps)
                def _():
                    idx_copy(step + 1, b1).wait()

                    @pl.when(step >= 1)
                    def _():
                        out_copy(step - 1, b1).wait()

                    gather_copy(b1).start()

                gather_copy(b).wait()
                out_copy(step, b).start()

                @pl.when(step + 2 < steps)
                def _():
                    idx_copy(step + 2, b).start()

        out_copy(steps - 1, (steps - 1) % 2).wait()

    return k(weight, idx)


def kernel(mask, weight):
    batch, hist = mask.shape
    _, dim = weight.shape
    num_indices = batch * hist
    idx = mask.reshape(num_indices)
    out = _sc_gather(weight, idx, num_indices, dim)
    return out.reshape(batch, hist, dim)


# speedup vs baseline: 2.3460x; 1.2519x over previous
import functools

import jax
import jax.numpy as jnp
from jax import lax
from jax.experimental import pallas as pl
from jax.experimental.pallas import tpu as pltpu
from jax.experimental.pallas import tpu_sc as plsc
from jax.experimental.layout import Format, Layout, with_layout_constraint

_NUM_CORES = 2
_NUM_SUBCORES = 16
_NUM_WORKERS = _NUM_CORES * _NUM_SUBCORES
_CHUNK = 800


def _sc_gather(weight, idx, num_indices, dim):
    b_per_w = num_indices // _NUM_WORKERS
    steps = b_per_w // _CHUNK
    mesh = plsc.VectorSubcoreMesh(core_axis_name="c", subcore_axis_name="s")

    @functools.partial(
        pl.kernel,
        mesh=mesh,
        compiler_params=pltpu.CompilerParams(use_tc_tiling_on_sc=False),
        out_type=jax.ShapeDtypeStruct((num_indices, dim), jnp.float32),
        scratch_types=[
            pltpu.VMEM((2, _CHUNK), jnp.int32),
            pltpu.VMEM((2, _CHUNK, dim), jnp.float32),
            pltpu.SemaphoreType.DMA((2,)),
            pltpu.SemaphoreType.DMA((2,)),
            pltpu.SemaphoreType.DMA((2,)),
        ],
    )
    def k(table_hbm, idx_hbm, out_hbm, idx_v, rows_v, sem_i, sem_g, sem_o):
        wid = lax.axis_index("s") * _NUM_CORES + lax.axis_index("c")
        base = wid * b_per_w

        def idx_copy(step, b):
            return pltpu.make_async_copy(
                idx_hbm.at[pl.ds(base + step * _CHUNK, _CHUNK)],
                idx_v.at[b],
                sem_i.at[b],
            )

        def gather_copy(b):
            return pltpu.make_async_copy(
                table_hbm.at[idx_v.at[b]], rows_v.at[b], sem_g.at[b]
            )

        def out_copy(step, b):
            return pltpu.make_async_copy(
                rows_v.at[b],
                out_hbm.at[pl.ds(base + step * _CHUNK, _CHUNK)],
                sem_o.at[b],
            )

        idx_copy(0, 0).start()
        idx_copy(1, 1).start()
        idx_copy(0, 0).wait()
        gather_copy(0).start()

        @pl.loop(0, steps, step=2)
        def _(i):
            for b in range(2):
                step = i + b
                b1 = 1 - b

                @pl.when(step + 1 < steps)
                def _():
                    idx_copy(step + 1, b1).wait()

                    @pl.when(step >= 1)
                    def _():
                        out_copy(step - 1, b1).wait()

                    gather_copy(b1).start()

                gather_copy(b).wait()
                out_copy(step, b).start()

                @pl.when(step + 2 < steps)
                def _():
                    idx_copy(step + 2, b).start()

        out_copy(steps - 1, (steps - 1) % 2).wait()

    return k(weight, idx)


def kernel(mask, weight):
    batch, hist = mask.shape
    _, dim = weight.shape
    num_indices = batch * hist
    idx = mask.reshape(num_indices)
    weight = with_layout_constraint(
        weight, Layout(major_to_minor=(0, 1), tiling=((8,),))
    )
    out = _sc_gather(weight, idx, num_indices, dim)
    return out.reshape(batch, hist, dim)
